# baseline (speedup 1.0000x reference)
import functools

import jax
import jax.numpy as jnp
from jax import lax
from jax.experimental import pallas as pl
from jax.experimental.pallas import tpu as pltpu
from jax.experimental.pallas import tpu_sc as plsc

N = 50000
E = 800000
F_IN = 16
H = 64
NP = 51200
RB = 1024
GRID = NP // RB
NB = NP // 32
NSC = NP // 2
DUMMY = NSC
ER = E // 128
ZR = 200
NEG_INF = float("-inf")


SUP = 4
ERP = ER + 8


def _sc_split(c, s, e16):
  rlo_sc = jnp.where(c == 0, 0, e16 // 128)
  rhi_sc = jnp.where(c == 0, (e16 + 127) // 128, ER)
  nrows = rhi_sc - rlo_sc
  per = nrows // 16
  rem = nrows % 16
  t_lo = rlo_sc + s * per + jnp.minimum(s, rem)
  t_cnt = per + jnp.where(s < rem, 1, 0)
  return t_lo, t_cnt


def _build_segsum(feat, split_sorted=True):
  mesh = plsc.VectorSubcoreMesh(core_axis_name="c", subcore_axis_name="s")

  def body(g_hbm, src_hbm, dst_hbm, eoff_hbm, out_hbm, eoffv, idxs, idxd,
           rows, stage, acc, sem_a, sem_b):
    c = lax.axis_index("c")
    s = lax.axis_index("s")
    base = c * NSC
    sems = (sem_a, sem_b)
    zvec = jnp.zeros((16,), jnp.float32)

    def zrow(i, carry):
      for k in range(feat // 16):
        stage[i, pl.ds(k * 16, 16)] = zvec
      return carry

    lax.fori_loop(0, ZR, zrow, 0)
    for j in range(NB // ZR):
      pltpu.sync_copy(stage, acc.at[pl.ds(s * NB + j * ZR, ZR)])

    @pl.when(s == 0)
    def _():
      pltpu.sync_copy(stage.at[pl.ds(0, 8)], acc.at[pl.ds(NSC, 8)])

    if split_sorted:
      pltpu.sync_copy(eoff_hbm, eoffv)
      e16 = eoffv[pl.ds(16, 16)][0]
      t_lo, t_cnt = _sc_split(c, s, e16)
    else:
      t_lo = s * (ER // 16) + jnp.minimum(s, ER % 16)
      t_cnt = (ER // 16) + jnp.where(s < ER % 16, 1, 0)
    ns = (t_cnt + SUP - 1) // SUP
    plsc.subcore_barrier()

    def load_idx(b, sc):
      r = t_lo + sc * SUP
      pltpu.sync_copy(src_hbm.at[pl.ds(r, SUP)], idxs.at[b])
      pltpu.sync_copy(dst_hbm.at[pl.ds(r, SUP)], idxd.at[b])
      for j in range(SUP):
        for k in range(8):
          sl = pl.ds(k * 16, 16)
          d = idxd[b, j, sl]
          inr = (d >= base) & (d < base + NSC)
          idxd[b, j, sl] = jnp.where(inr, d - base, DUMMY)

    def fire(b):
      for j in range(SUP):
        pltpu.async_copy(g_hbm.at[idxs.at[b].at[j]],
                         rows.at[b].at[pl.ds(j * 128, 128)], sems[b])

    def drain(b):
      for j in range(SUP):
        pltpu.make_async_copy(g_hbm.at[idxs.at[b].at[j]],
                              rows.at[b].at[pl.ds(j * 128, 128)],
                              sems[b]).wait()

    def scatter(b, sc):
      left = t_cnt - sc * SUP
      for j in range(SUP):
        @pl.when(j < left)
        def _(j=j):
          pltpu.sync_copy(rows.at[b].at[pl.ds(j * 128, 128)],
                          acc.at[idxd.at[b].at[j]], add=True)

    @pl.when(ns > 0)
    def _():
      load_idx(0, 0)
      fire(0)

    def pair(p, carry):
      for b in range(2):
        sc = 2 * p + b

        @pl.when(sc < ns)
        def _(b=b, sc=sc):
          @pl.when(sc + 1 < ns)
          def _():
            load_idx(1 - b, sc + 1)
            fire(1 - b)

          drain(b)
          scatter(b, sc)
      return carry

    lax.fori_loop(0, (ns + 1) // 2, pair, 0)
    plsc.subcore_barrier()

    for j in range(NB // ZR):
      off = s * NB + j * ZR
      pltpu.sync_copy(acc.at[pl.ds(off, ZR)], stage)
      pltpu.sync_copy(stage, out_hbm.at[pl.ds(base + off, ZR)])

  return pl.kernel(
      body,
      out_type=jax.ShapeDtypeStruct((NP, feat), jnp.float32),
      mesh=mesh,
      compiler_params=pltpu.CompilerParams(use_tc_tiling_on_sc=False),
      scratch_types=[
          pltpu.VMEM((48,), jnp.int32),
          pltpu.VMEM((2, SUP, 128), jnp.int32),
          pltpu.VMEM((2, SUP, 128), jnp.int32),
          pltpu.VMEM((2, SUP * 128, feat), jnp.float32),
          pltpu.VMEM((ZR, feat), jnp.float32),
          pltpu.VMEM_SHARED((NSC + 8, feat), jnp.float32),
          pltpu.SemaphoreType.DMA,
          pltpu.SemaphoreType.DMA,
      ],
  )


def _build_segmax():
  mesh = plsc.VectorSubcoreMesh(core_axis_name="c", subcore_axis_name="s")

  def body(a_hbm, src_hbm, dst_hbm, eoff_hbm, out_hbm, eoff, idxs, idxd,
           rows, acc, sem_a, sem_b):
    c = lax.axis_index("c")
    s = lax.axis_index("s")
    tg = c * 16 + s
    nbase = tg * NB
    sems = (sem_a, sem_b)
    pltpu.sync_copy(eoff_hbm, eoff)
    lo = eoff[pl.ds(tg, 16)][0]
    hi = eoff[pl.ds(tg + 1, 16)][0]
    ninf = jnp.full((16,), NEG_INF, jnp.float32)

    def irow(i, carry):
      for k in range(4):
        acc[i, pl.ds(k * 16, 16)] = ninf
      return carry

    lax.fori_loop(0, NB, irow, 0)
    rlo = lo // 128
    rhi = (hi + 127) // 128
    nr = rhi - rlo

    def load_idx(b, r):
      pltpu.sync_copy(src_hbm.at[pl.ds(r, 1)], idxs.at[b])
      pltpu.sync_copy(dst_hbm.at[pl.ds(r, 1)],
                      idxd.at[b].at[:, pl.ds(0, 128)])

    def fire(b):
      pltpu.async_copy(a_hbm.at[idxs.at[b].at[0]], rows.at[b], sems[b])

    def drain(b):
      pltpu.make_async_copy(a_hbm.at[idxs.at[b].at[0]], rows.at[b],
                            sems[b]).wait()

    def flush(rl_cur, avs):
      for k, ak in enumerate(avs):
        sl = pl.ds(k * 16, 16)
        acc[rl_cur, sl] = jnp.maximum(acc[rl_cur, sl], ak)

    def rmw(b, r):
      jlo = jnp.maximum(lo - r * 128, 0)
      jhi = jnp.minimum(hi - r * 128, 128)

      def edge(j, carry):
        rl_cur = carry[0]
        avs = carry[1:]
        rl = idxd[b, 0, pl.ds(j, 16)][0] - nbase
        new = rl != rl_cur

        @pl.when(new & (rl_cur >= 0))
        def _():
          flush(rl_cur, avs)

        out = [rl]
        for k, ak in enumerate(avs):
          rv = rows[b, j, pl.ds(k * 16, 16)]
          out.append(jnp.maximum(jnp.where(new, ninf, ak), rv))
        return tuple(out)

      carry0 = (jnp.int32(-1), ninf, ninf, ninf, ninf)
      fin = lax.fori_loop(jlo, jhi, edge, carry0)

      @pl.when(fin[0] >= 0)
      def _():
        flush(fin[0], fin[1:])

    @pl.when(nr > 0)
    def _():
      load_idx(0, rlo)
      fire(0)

    def pair(p, carry):
      for b in range(2):
        r = rlo + 2 * p + b

        @pl.when(r < rhi)
        def _(b=b, r=r):
          @pl.when(r + 1 < rhi)
          def _():
            load_idx(1 - b, r + 1)
            fire(1 - b)

          drain(b)
          rmw(b, r)
      return carry

    lax.fori_loop(0, (nr + 1) // 2, pair, 0)
    pltpu.sync_copy(acc, out_hbm.at[pl.ds(nbase, NB)])

  return pl.kernel(
      body,
      out_type=jax.ShapeDtypeStruct((NP, H), jnp.float32),
      mesh=mesh,
      compiler_params=pltpu.CompilerParams(use_tc_tiling_on_sc=False),
      scratch_types=[
          pltpu.VMEM((48,), jnp.int32),
          pltpu.VMEM((2, 1, 128), jnp.int32),
          pltpu.VMEM((2, 1, 144), jnp.int32),
          pltpu.VMEM((2, 128, H), jnp.float32),
          pltpu.VMEM((NB, H), jnp.float32),
          pltpu.SemaphoreType.DMA,
          pltpu.SemaphoreType.DMA,
      ],
  )


_segsum16u = _build_segsum(16, split_sorted=False)
_segsum32 = _build_segsum(32)
_segmax = _build_segmax()


def _segsum64(g, bsrc2, bdst2, eoff48):
  lo = _segsum32(g[:, :32], bsrc2, bdst2, eoff48)
  hi = _segsum32(g[:, 32:], bsrc2, bdst2, eoff48)
  return jnp.concatenate([lo, hi], axis=1)


def _rowspec(feat):
  return pl.BlockSpec((RB, feat), lambda i: (i, 0))


def _fullspec(shape):
  nd = len(shape)
  return pl.BlockSpec(shape, lambda i, _nd=nd: (0,) * nd)


def _tc_prep(deg16):
  def body(deg_ref, nrm_ref, nsq_ref):
    d = deg_ref[...][:, 0:1]
    nrm = jnp.where(d > 0, lax.rsqrt(jnp.maximum(d, 1.0)), 0.0)
    nrm_ref[...] = nrm
    nsq_ref[...] = nrm * nrm

  return pl.pallas_call(
      body,
      grid=(GRID,),
      in_specs=[_rowspec(16)],
      out_specs=[_rowspec(1), _rowspec(1)],
      out_shape=[jax.ShapeDtypeStruct((NP, 1), jnp.float32)] * 2,
  )(deg16)


def _tc_scale(x, nrm):
  feat = x.shape[1]

  def body(x_ref, nrm_ref, g_ref):
    g_ref[...] = x_ref[...] * nrm_ref[...]

  return pl.pallas_call(
      body,
      grid=(GRID,),
      in_specs=[_rowspec(feat), _rowspec(1)],
      out_specs=_rowspec(feat),
      out_shape=jax.ShapeDtypeStruct((NP, feat), jnp.float32),
  )(x, nrm)


def _tc_mid(s1, nrm, nsq):
  feat = s1.shape[1]

  def body(s_ref, nrm_ref, nsq_ref, xt1_ref, g2_ref):
    sv = s_ref[...]
    xt1_ref[...] = -(sv * nrm_ref[...])
    g2_ref[...] = -(sv * nsq_ref[...])

  return pl.pallas_call(
      body,
      grid=(GRID,),
      in_specs=[_rowspec(feat), _rowspec(1), _rowspec(1)],
      out_specs=[_rowspec(feat), _rowspec(feat)],
      out_shape=[jax.ShapeDtypeStruct((NP, feat), jnp.float32)] * 2,
  )(s1, nrm, nsq)


def _tc_cheb_edge(t0, xt1, s2, nrm, w0, w1, w2, b, twt, pwt, tb, pb):
  feat = t0.shape[1]

  def body(t0_ref, xt1_ref, s2_ref, nrm_ref, w0_ref, w1_ref, w2_ref, b_ref,
           twt_ref, pwt_ref, tb_ref, pb_ref, a_ref, c_ref):
    t0v = t0_ref[...]
    xt2 = -2.0 * (s2_ref[...] * nrm_ref[...]) - t0v
    h = (jnp.dot(t0v, w0_ref[...], preferred_element_type=jnp.float32)
         + jnp.dot(xt1_ref[...], w1_ref[...],
                   preferred_element_type=jnp.float32)
         + jnp.dot(xt2, w2_ref[...], preferred_element_type=jnp.float32)
         + b_ref[...])
    h = jnp.maximum(h, 0.0)
    a_ref[...] = jnp.dot(h, twt_ref[...], preferred_element_type=jnp.float32)
    c_ref[...] = (jnp.dot(h, pwt_ref[...] - twt_ref[...],
                          preferred_element_type=jnp.float32)
                  + tb_ref[...] + pb_ref[...])

  return pl.pallas_call(
      body,
      grid=(GRID,),
      in_specs=[_rowspec(feat), _rowspec(feat), _rowspec(feat), _rowspec(1),
                _fullspec((feat, H)), _fullspec((feat, H)),
                _fullspec((feat, H)), _fullspec((1, H)),
                _fullspec((H, H)), _fullspec((H, H)),
                _fullspec((1, H)), _fullspec((1, H))],
      out_specs=[_rowspec(H), _rowspec(H)],
      out_shape=[jax.ShapeDtypeStruct((NP, H), jnp.float32)] * 2,
  )(t0, xt1, s2, nrm, w0, w1, w2, b, twt, pwt, tb, pb)


def _tc_edge_post(m, cc, nrm):
  def body(m_ref, c_ref, nrm_ref, h_ref, g_ref):
    nc = nrm_ref[...]
    h = jnp.where(nc > 0, jnp.maximum(m_ref[...] + c_ref[...], 0.0), 0.0)
    h_ref[...] = h
    g_ref[...] = h * nc

  return pl.pallas_call(
      body,
      grid=(GRID,),
      in_specs=[_rowspec(H), _rowspec(H), _rowspec(1)],
      out_specs=[_rowspec(H), _rowspec(H)],
      out_shape=[jax.ShapeDtypeStruct((NP, H), jnp.float32)] * 2,
  )(m, cc, nrm)


def _tc_cheb_final(t0, xt1, s2, nrm, w0, w1, w2, b):
  def body(t0_ref, xt1_ref, s2_ref, nrm_ref, w0_ref, w1_ref, w2_ref, b_ref,
           o_ref):
    i = pl.program_id(0)
    t0v = t0_ref[...]
    xt2 = -2.0 * (s2_ref[...] * nrm_ref[...]) - t0v
    h = (jnp.dot(t0v, w0_ref[...], preferred_element_type=jnp.float32)
         + jnp.dot(xt1_ref[...], w1_ref[...],
                   preferred_element_type=jnp.float32)
         + jnp.dot(xt2, w2_ref[...], preferred_element_type=jnp.float32)
         + b_ref[...])
    h = jnp.maximum(h, 0.0)
    ridx = lax.broadcasted_iota(jnp.int32, (RB, 1), 0) + i * RB
    h = jnp.where(ridx < N, h, 0.0)

    @pl.when(i == 0)
    def _():
      o_ref[...] = jnp.zeros_like(o_ref)

    o_ref[...] += jnp.sum(h, axis=0, keepdims=True)

    @pl.when(i == GRID - 1)
    def _():
      o_ref[...] *= (1.0 / N)

  return pl.pallas_call(
      body,
      grid=(GRID,),
      in_specs=[_rowspec(H), _rowspec(H), _rowspec(H), _rowspec(1),
                _fullspec((H, H)), _fullspec((H, H)), _fullspec((H, H)),
                _fullspec((1, H))],
      out_specs=pl.BlockSpec((1, H), lambda i: (0, 0)),
      out_shape=jax.ShapeDtypeStruct((1, H), jnp.float32),
  )(t0, xt1, s2, nrm, w0, w1, w2, b)


def kernel(x, edge_index, W1, b1, W2, b2, W3, b3,
           tw1, tb1, pw1, pb1, tw2, tb2, pw2, pb2):
  src = edge_index[0]
  dst = edge_index[1]

  npad = ERP * 128 - E
  rsrc2 = jnp.concatenate(
      [src, jnp.zeros((npad,), jnp.int32)]).reshape(ERP, 128)
  rdst2 = jnp.concatenate(
      [dst, jnp.full((npad,), NP, jnp.int32)]).reshape(ERP, 128)
  key = (dst.astype(jnp.uint32) << 16) | src.astype(jnp.uint32)
  skey = lax.sort(key, is_stable=False)
  bsrc2 = jnp.concatenate(
      [(skey & 0xFFFF).astype(jnp.int32),
       jnp.zeros((npad,), jnp.int32)]).reshape(ERP, 128)
  bdst2 = jnp.concatenate(
      [(skey >> 16).astype(jnp.int32),
       jnp.full((npad,), NP, jnp.int32)]).reshape(ERP, 128)
  bounds = (jnp.arange(33, dtype=jnp.uint32) * NB) << 16
  eoff = jnp.searchsorted(skey, bounds).astype(jnp.int32)
  eoff48 = jnp.concatenate([eoff, jnp.full((15,), E, jnp.int32)])

  x_p = jnp.pad(x, ((0, NP - N), (0, 0)))
  b1r = b1.reshape(1, H)
  b2r = b2.reshape(1, H)
  b3r = b3.reshape(1, H)
  tb1r = tb1.reshape(1, H)
  pb1r = pb1.reshape(1, H)
  tb2r = tb2.reshape(1, H)
  pb2r = pb2.reshape(1, H)

  ones16 = jnp.ones((NP, 16), jnp.float32)
  deg16 = _segsum16u(ones16, rsrc2, rdst2, eoff48)
  nrm, nsq = _tc_prep(deg16)

  g1 = _tc_scale(x_p, nrm)
  s1 = _segsum16u(g1, rsrc2, rdst2, eoff48)
  xt1, g2 = _tc_mid(s1, nrm, nsq)
  s2 = _segsum16u(g2, rsrc2, rdst2, eoff48)
  a1, c1 = _tc_cheb_edge(x_p, xt1, s2, nrm, W1[0], W1[1], W1[2], b1r,
                         tw1.T, pw1.T, tb1r, pb1r)
  m1 = _segmax(a1, bsrc2, bdst2, eoff48)
  h2, g3 = _tc_edge_post(m1, c1, nrm)

  s3 = _segsum64(g3, bsrc2, bdst2, eoff48)
  xt1b, g4 = _tc_mid(s3, nrm, nsq)
  s4 = _segsum64(g4, bsrc2, bdst2, eoff48)
  a2, c2 = _tc_cheb_edge(h2, xt1b, s4, nrm, W2[0], W2[1], W2[2], b2r,
                         tw2.T, pw2.T, tb2r, pb2r)
  m2 = _segmax(a2, bsrc2, bdst2, eoff48)
  h4, g5 = _tc_edge_post(m2, c2, nrm)

  s5 = _segsum64(g5, bsrc2, bdst2, eoff48)
  xt1c, g6 = _tc_mid(s5, nrm, nsq)
  s6 = _segsum64(g6, bsrc2, bdst2, eoff48)
  return _tc_cheb_final(h4, xt1c, s6, nrm, W3[0], W3[1], W3[2], b3r)


# speedup vs baseline: 1.1233x; 1.1233x over previous
import functools

import jax
import jax.numpy as jnp
from jax import lax
from jax.experimental import pallas as pl
from jax.experimental.pallas import tpu as pltpu
from jax.experimental.pallas import tpu_sc as plsc

N = 50000
E = 800000
F_IN = 16
H = 64
NP = 51200
RB = 1024
GRID = NP // RB
NB = NP // 32
NSC = NP // 2
DUMMY = NSC
ER = E // 128
ZR = 200
NEG_INF = float("-inf")


SUP = 4
ERP = ER + 8


def _sc_split(c, s, e16):
  rlo_sc = jnp.where(c == 0, 0, e16 // 128)
  rhi_sc = jnp.where(c == 0, (e16 + 127) // 128, ER)
  nrows = rhi_sc - rlo_sc
  per = nrows // 16
  rem = nrows % 16
  t_lo = rlo_sc + s * per + jnp.minimum(s, rem)
  t_cnt = per + jnp.where(s < rem, 1, 0)
  return t_lo, t_cnt


def _build_segsum(feat, split_sorted=True):
  mesh = plsc.VectorSubcoreMesh(core_axis_name="c", subcore_axis_name="s")

  def body(g_hbm, src_hbm, dst_hbm, eoff_hbm, out_hbm, eoffv, idxs, idxd,
           rows, stage, acc, sem_a, sem_b):
    c = lax.axis_index("c")
    s = lax.axis_index("s")
    base = c * NSC
    sems = (sem_a, sem_b)
    zvec = jnp.zeros((16,), jnp.float32)

    def zrow(i, carry):
      for k in range(feat // 16):
        stage[i, pl.ds(k * 16, 16)] = zvec
      return carry

    lax.fori_loop(0, ZR, zrow, 0)
    for j in range(NB // ZR):
      pltpu.sync_copy(stage, acc.at[pl.ds(s * NB + j * ZR, ZR)])

    @pl.when(s == 0)
    def _():
      pltpu.sync_copy(stage.at[pl.ds(0, 8)], acc.at[pl.ds(NSC, 8)])

    if split_sorted:
      pltpu.sync_copy(eoff_hbm, eoffv)
      e16 = eoffv[pl.ds(16, 16)][0]
      t_lo, t_cnt = _sc_split(c, s, e16)
    else:
      t_lo = s * (ER // 16) + jnp.minimum(s, ER % 16)
      t_cnt = (ER // 16) + jnp.where(s < ER % 16, 1, 0)
    ns = (t_cnt + SUP - 1) // SUP
    plsc.subcore_barrier()

    def load_idx(b, sc):
      r = t_lo + sc * SUP
      pltpu.sync_copy(src_hbm.at[pl.ds(r, SUP)], idxs.at[b])
      pltpu.sync_copy(dst_hbm.at[pl.ds(r, SUP)], idxd.at[b])
      for j in range(SUP):
        for k in range(8):
          sl = pl.ds(k * 16, 16)
          d = idxd[b, j, sl]
          inr = (d >= base) & (d < base + NSC)
          idxd[b, j, sl] = jnp.where(inr, d - base, DUMMY)

    def fire(b):
      for j in range(SUP):
        pltpu.async_copy(g_hbm.at[idxs.at[b].at[j]],
                         rows.at[b].at[pl.ds(j * 128, 128)], sems[b])

    def drain(b):
      for j in range(SUP):
        pltpu.make_async_copy(g_hbm.at[idxs.at[b].at[j]],
                              rows.at[b].at[pl.ds(j * 128, 128)],
                              sems[b]).wait()

    def scatter(b, sc):
      left = t_cnt - sc * SUP
      for j in range(SUP):
        @pl.when(j < left)
        def _(j=j):
          pltpu.sync_copy(rows.at[b].at[pl.ds(j * 128, 128)],
                          acc.at[idxd.at[b].at[j]], add=True)

    @pl.when(ns > 0)
    def _():
      load_idx(0, 0)
      fire(0)

    def pair(p, carry):
      for b in range(2):
        sc = 2 * p + b

        @pl.when(sc < ns)
        def _(b=b, sc=sc):
          @pl.when(sc + 1 < ns)
          def _():
            load_idx(1 - b, sc + 1)
            fire(1 - b)

          drain(b)
          scatter(b, sc)
      return carry

    lax.fori_loop(0, (ns + 1) // 2, pair, 0)
    plsc.subcore_barrier()

    for j in range(NB // ZR):
      off = s * NB + j * ZR
      pltpu.sync_copy(acc.at[pl.ds(off, ZR)], stage)
      pltpu.sync_copy(stage, out_hbm.at[pl.ds(base + off, ZR)])

  return pl.kernel(
      body,
      out_type=jax.ShapeDtypeStruct((NP, feat), jnp.float32),
      mesh=mesh,
      compiler_params=pltpu.CompilerParams(use_tc_tiling_on_sc=False),
      scratch_types=[
          pltpu.VMEM((48,), jnp.int32),
          pltpu.VMEM((2, SUP, 128), jnp.int32),
          pltpu.VMEM((2, SUP, 128), jnp.int32),
          pltpu.VMEM((2, SUP * 128, feat), jnp.float32),
          pltpu.VMEM((ZR, feat), jnp.float32),
          pltpu.VMEM_SHARED((NSC + 8, feat), jnp.float32),
          pltpu.SemaphoreType.DMA,
          pltpu.SemaphoreType.DMA,
      ],
  )


def _build_segmax():
  mesh = plsc.VectorSubcoreMesh(core_axis_name="c", subcore_axis_name="s")

  def body(a_hbm, src_hbm, dst_hbm, eoff_hbm, out_hbm, eoff, idxs, idxd,
           rows, acc, sem_a, sem_b):
    c = lax.axis_index("c")
    s = lax.axis_index("s")
    tg = c * 16 + s
    nbase = tg * NB
    sems = (sem_a, sem_b)
    pltpu.sync_copy(eoff_hbm, eoff)
    lo = eoff[pl.ds(tg, 16)][0]
    hi = eoff[pl.ds(tg + 1, 16)][0]
    ninf = jnp.full((16,), NEG_INF, jnp.float32)

    def irow(i, carry):
      for k in range(4):
        acc[i, pl.ds(k * 16, 16)] = ninf
      return carry

    lax.fori_loop(0, NB, irow, 0)
    rlo = lo // 128
    rhi = (hi + 127) // 128
    nr = rhi - rlo

    def load_idx(b, r):
      pltpu.sync_copy(src_hbm.at[pl.ds(r, 1)], idxs.at[b])
      pltpu.sync_copy(dst_hbm.at[pl.ds(r, 1)],
                      idxd.at[b].at[:, pl.ds(0, 128)])

    def fire(b):
      pltpu.async_copy(a_hbm.at[idxs.at[b].at[0]], rows.at[b], sems[b])

    def drain(b):
      pltpu.make_async_copy(a_hbm.at[idxs.at[b].at[0]], rows.at[b],
                            sems[b]).wait()

    def flush(rl_cur, avs):
      for k, ak in enumerate(avs):
        sl = pl.ds(k * 16, 16)
        acc[rl_cur, sl] = jnp.maximum(acc[rl_cur, sl], ak)

    def rmw(b, r):
      jlo = jnp.maximum(lo - r * 128, 0)
      jhi = jnp.minimum(hi - r * 128, 128)

      def edge(j, carry):
        rl_cur = carry[0]
        avs = carry[1:]
        rl = idxd[b, 0, pl.ds(j, 16)][0] - nbase
        new = rl != rl_cur

        @pl.when(new & (rl_cur >= 0))
        def _():
          flush(rl_cur, avs)

        out = [rl]
        for k, ak in enumerate(avs):
          rv = rows[b, j, pl.ds(k * 16, 16)]
          out.append(jnp.maximum(jnp.where(new, ninf, ak), rv))
        return tuple(out)

      carry0 = (jnp.int32(-1), ninf, ninf, ninf, ninf)
      fin = lax.fori_loop(jlo, jhi, edge, carry0)

      @pl.when(fin[0] >= 0)
      def _():
        flush(fin[0], fin[1:])

    @pl.when(nr > 0)
    def _():
      load_idx(0, rlo)
      fire(0)

    def pair(p, carry):
      for b in range(2):
        r = rlo + 2 * p + b

        @pl.when(r < rhi)
        def _(b=b, r=r):
          @pl.when(r + 1 < rhi)
          def _():
            load_idx(1 - b, r + 1)
            fire(1 - b)

          drain(b)
          rmw(b, r)
      return carry

    lax.fori_loop(0, (nr + 1) // 2, pair, 0)
    pltpu.sync_copy(acc, out_hbm.at[pl.ds(nbase, NB)])

  return pl.kernel(
      body,
      out_type=jax.ShapeDtypeStruct((NP, H), jnp.float32),
      mesh=mesh,
      compiler_params=pltpu.CompilerParams(use_tc_tiling_on_sc=False),
      scratch_types=[
          pltpu.VMEM((48,), jnp.int32),
          pltpu.VMEM((2, 1, 128), jnp.int32),
          pltpu.VMEM((2, 1, 144), jnp.int32),
          pltpu.VMEM((2, 128, H), jnp.float32),
          pltpu.VMEM((NB, H), jnp.float32),
          pltpu.SemaphoreType.DMA,
          pltpu.SemaphoreType.DMA,
      ],
  )


_segsum16 = _build_segsum(16)
_segsum32 = _build_segsum(32)
_segmax = _build_segmax()


def _segsum64(g, bsrc2, bdst2, eoff48):
  lo = _segsum32(g[:, :32], bsrc2, bdst2, eoff48)
  hi = _segsum32(g[:, 32:], bsrc2, bdst2, eoff48)
  return jnp.concatenate([lo, hi], axis=1)


def _rowspec(feat):
  return pl.BlockSpec((RB, feat), lambda i: (i, 0))


def _fullspec(shape):
  nd = len(shape)
  return pl.BlockSpec(shape, lambda i, _nd=nd: (0,) * nd)


def _tc_prep(deg16):
  def body(deg_ref, nrm_ref, nsq_ref):
    d = deg_ref[...][:, 0:1]
    nrm = jnp.where(d > 0, lax.rsqrt(jnp.maximum(d, 1.0)), 0.0)
    nrm_ref[...] = nrm
    nsq_ref[...] = nrm * nrm

  return pl.pallas_call(
      body,
      grid=(GRID,),
      in_specs=[_rowspec(16)],
      out_specs=[_rowspec(1), _rowspec(1)],
      out_shape=[jax.ShapeDtypeStruct((NP, 1), jnp.float32)] * 2,
  )(deg16)


def _tc_scale(x, nrm):
  feat = x.shape[1]

  def body(x_ref, nrm_ref, g_ref):
    g_ref[...] = x_ref[...] * nrm_ref[...]

  return pl.pallas_call(
      body,
      grid=(GRID,),
      in_specs=[_rowspec(feat), _rowspec(1)],
      out_specs=_rowspec(feat),
      out_shape=jax.ShapeDtypeStruct((NP, feat), jnp.float32),
  )(x, nrm)


def _tc_mid(s1, nrm, nsq):
  feat = s1.shape[1]

  def body(s_ref, nrm_ref, nsq_ref, xt1_ref, g2_ref):
    sv = s_ref[...]
    xt1_ref[...] = -(sv * nrm_ref[...])
    g2_ref[...] = -(sv * nsq_ref[...])

  return pl.pallas_call(
      body,
      grid=(GRID,),
      in_specs=[_rowspec(feat), _rowspec(1), _rowspec(1)],
      out_specs=[_rowspec(feat), _rowspec(feat)],
      out_shape=[jax.ShapeDtypeStruct((NP, feat), jnp.float32)] * 2,
  )(s1, nrm, nsq)


def _tc_cheb_edge(t0, xt1, s2, nrm, w0, w1, w2, b, twt, pwt, tb, pb):
  feat = t0.shape[1]

  def body(t0_ref, xt1_ref, s2_ref, nrm_ref, w0_ref, w1_ref, w2_ref, b_ref,
           twt_ref, pwt_ref, tb_ref, pb_ref, a_ref, c_ref):
    t0v = t0_ref[...]
    xt2 = -2.0 * (s2_ref[...] * nrm_ref[...]) - t0v
    h = (jnp.dot(t0v, w0_ref[...], preferred_element_type=jnp.float32)
         + jnp.dot(xt1_ref[...], w1_ref[...],
                   preferred_element_type=jnp.float32)
         + jnp.dot(xt2, w2_ref[...], preferred_element_type=jnp.float32)
         + b_ref[...])
    h = jnp.maximum(h, 0.0)
    a_ref[...] = jnp.dot(h, twt_ref[...], preferred_element_type=jnp.float32)
    c_ref[...] = (jnp.dot(h, pwt_ref[...] - twt_ref[...],
                          preferred_element_type=jnp.float32)
                  + tb_ref[...] + pb_ref[...])

  return pl.pallas_call(
      body,
      grid=(GRID,),
      in_specs=[_rowspec(feat), _rowspec(feat), _rowspec(feat), _rowspec(1),
                _fullspec((feat, H)), _fullspec((feat, H)),
                _fullspec((feat, H)), _fullspec((1, H)),
                _fullspec((H, H)), _fullspec((H, H)),
                _fullspec((1, H)), _fullspec((1, H))],
      out_specs=[_rowspec(H), _rowspec(H)],
      out_shape=[jax.ShapeDtypeStruct((NP, H), jnp.float32)] * 2,
  )(t0, xt1, s2, nrm, w0, w1, w2, b, twt, pwt, tb, pb)


def _tc_edge_post(m, cc, nrm):
  def body(m_ref, c_ref, nrm_ref, h_ref, g_ref):
    nc = nrm_ref[...]
    h = jnp.where(nc > 0, jnp.maximum(m_ref[...] + c_ref[...], 0.0), 0.0)
    h_ref[...] = h
    g_ref[...] = h * nc

  return pl.pallas_call(
      body,
      grid=(GRID,),
      in_specs=[_rowspec(H), _rowspec(H), _rowspec(1)],
      out_specs=[_rowspec(H), _rowspec(H)],
      out_shape=[jax.ShapeDtypeStruct((NP, H), jnp.float32)] * 2,
  )(m, cc, nrm)


def _tc_cheb_final(t0, xt1, s2, nrm, w0, w1, w2, b):
  def body(t0_ref, xt1_ref, s2_ref, nrm_ref, w0_ref, w1_ref, w2_ref, b_ref,
           o_ref):
    i = pl.program_id(0)
    t0v = t0_ref[...]
    xt2 = -2.0 * (s2_ref[...] * nrm_ref[...]) - t0v
    h = (jnp.dot(t0v, w0_ref[...], preferred_element_type=jnp.float32)
         + jnp.dot(xt1_ref[...], w1_ref[...],
                   preferred_element_type=jnp.float32)
         + jnp.dot(xt2, w2_ref[...], preferred_element_type=jnp.float32)
         + b_ref[...])
    h = jnp.maximum(h, 0.0)
    ridx = lax.broadcasted_iota(jnp.int32, (RB, 1), 0) + i * RB
    h = jnp.where(ridx < N, h, 0.0)

    @pl.when(i == 0)
    def _():
      o_ref[...] = jnp.zeros_like(o_ref)

    o_ref[...] += jnp.sum(h, axis=0, keepdims=True)

    @pl.when(i == GRID - 1)
    def _():
      o_ref[...] *= (1.0 / N)

  return pl.pallas_call(
      body,
      grid=(GRID,),
      in_specs=[_rowspec(H), _rowspec(H), _rowspec(H), _rowspec(1),
                _fullspec((H, H)), _fullspec((H, H)), _fullspec((H, H)),
                _fullspec((1, H))],
      out_specs=pl.BlockSpec((1, H), lambda i: (0, 0)),
      out_shape=jax.ShapeDtypeStruct((1, H), jnp.float32),
  )(t0, xt1, s2, nrm, w0, w1, w2, b)


def kernel(x, edge_index, W1, b1, W2, b2, W3, b3,
           tw1, tb1, pw1, pb1, tw2, tb2, pw2, pb2):
  src = edge_index[0]
  dst = edge_index[1]

  npad = ERP * 128 - E
  key = (dst.astype(jnp.uint32) << 16) | src.astype(jnp.uint32)
  skey = jnp.sort(key)
  bsrc2 = jnp.concatenate(
      [(skey & 0xFFFF).astype(jnp.int32),
       jnp.zeros((npad,), jnp.int32)]).reshape(ERP, 128)
  bdst2 = jnp.concatenate(
      [(skey >> 16).astype(jnp.int32),
       jnp.full((npad,), NP, jnp.int32)]).reshape(ERP, 128)
  bounds = (jnp.arange(33, dtype=jnp.uint32) * NB) << 16
  eoff = jnp.searchsorted(skey, bounds).astype(jnp.int32)
  eoff48 = jnp.concatenate([eoff, jnp.full((15,), E, jnp.int32)])

  x_p = jnp.pad(x, ((0, NP - N), (0, 0)))
  b1r = b1.reshape(1, H)
  b2r = b2.reshape(1, H)
  b3r = b3.reshape(1, H)
  tb1r = tb1.reshape(1, H)
  pb1r = pb1.reshape(1, H)
  tb2r = tb2.reshape(1, H)
  pb2r = pb2.reshape(1, H)

  ones16 = jnp.ones((NP, 16), jnp.float32)
  deg16 = _segsum16(ones16, bsrc2, bdst2, eoff48)
  nrm, nsq = _tc_prep(deg16)

  g1 = _tc_scale(x_p, nrm)
  s1 = _segsum16(g1, bsrc2, bdst2, eoff48)
  xt1, g2 = _tc_mid(s1, nrm, nsq)
  s2 = _segsum16(g2, bsrc2, bdst2, eoff48)
  a1, c1 = _tc_cheb_edge(x_p, xt1, s2, nrm, W1[0], W1[1], W1[2], b1r,
                         tw1.T, pw1.T, tb1r, pb1r)
  m1 = _segmax(a1, bsrc2, bdst2, eoff48)
  h2, g3 = _tc_edge_post(m1, c1, nrm)

  s3 = _segsum64(g3, bsrc2, bdst2, eoff48)
  xt1b, g4 = _tc_mid(s3, nrm, nsq)
  s4 = _segsum64(g4, bsrc2, bdst2, eoff48)
  a2, c2 = _tc_cheb_edge(h2, xt1b, s4, nrm, W2[0], W2[1], W2[2], b2r,
                         tw2.T, pw2.T, tb2r, pb2r)
  m2 = _segmax(a2, bsrc2, bdst2, eoff48)
  h4, g5 = _tc_edge_post(m2, c2, nrm)

  s5 = _segsum64(g5, bsrc2, bdst2, eoff48)
  xt1c, g6 = _tc_mid(s5, nrm, nsq)
  s6 = _segsum64(g6, bsrc2, bdst2, eoff48)
  return _tc_cheb_final(h4, xt1c, s6, nrm, W3[0], W3[1], W3[2], b3r)


# speedup vs baseline: 1.1562x; 1.0292x over previous
import functools

import jax
import jax.numpy as jnp
from jax import lax
from jax.experimental import pallas as pl
from jax.experimental.pallas import tpu as pltpu
from jax.experimental.pallas import tpu_sc as plsc

N = 50000
E = 800000
F_IN = 16
H = 64
NP = 51200
RB = 2048
GRID = NP // RB
NB = NP // 32
NSC = NP // 2
DUMMY = NSC
ER = E // 128
ZR = 200
NEG_INF = float("-inf")


SUP = 4
ERP = ER + 8


def _sc_split(c, s, e16):
  rlo_sc = jnp.where(c == 0, 0, e16 // 128)
  rhi_sc = jnp.where(c == 0, (e16 + 127) // 128, ER)
  nrows = rhi_sc - rlo_sc
  per = nrows // 16
  rem = nrows % 16
  t_lo = rlo_sc + s * per + jnp.minimum(s, rem)
  t_cnt = per + jnp.where(s < rem, 1, 0)
  return t_lo, t_cnt


def _build_segsum(feat, split_sorted=True):
  mesh = plsc.VectorSubcoreMesh(core_axis_name="c", subcore_axis_name="s")

  def body(g_hbm, src_hbm, dst_hbm, eoff_hbm, out_hbm, eoffv, idxs, idxd,
           rows, stage, acc, sem_a, sem_b):
    c = lax.axis_index("c")
    s = lax.axis_index("s")
    base = c * NSC
    sems = (sem_a, sem_b)
    zvec = jnp.zeros((16,), jnp.float32)

    def zrow(i, carry):
      for k in range(feat // 16):
        stage[i, pl.ds(k * 16, 16)] = zvec
      return carry

    lax.fori_loop(0, ZR, zrow, 0)
    for j in range(NB // ZR):
      pltpu.sync_copy(stage, acc.at[pl.ds(s * NB + j * ZR, ZR)])

    @pl.when(s == 0)
    def _():
      pltpu.sync_copy(stage.at[pl.ds(0, 8)], acc.at[pl.ds(NSC, 8)])

    if split_sorted:
      pltpu.sync_copy(eoff_hbm, eoffv)
      e16 = eoffv[pl.ds(16, 16)][0]
      t_lo, t_cnt = _sc_split(c, s, e16)
    else:
      t_lo = s * (ER // 16) + jnp.minimum(s, ER % 16)
      t_cnt = (ER // 16) + jnp.where(s < ER % 16, 1, 0)
    ns = (t_cnt + SUP - 1) // SUP
    plsc.subcore_barrier()

    def load_idx(b, sc):
      r = t_lo + sc * SUP
      pltpu.sync_copy(src_hbm.at[pl.ds(r, SUP)], idxs.at[b])
      pltpu.sync_copy(dst_hbm.at[pl.ds(r, SUP)], idxd.at[b])
      for j in range(SUP):
        for k in range(8):
          sl = pl.ds(k * 16, 16)
          d = idxd[b, j, sl]
          inr = (d >= base) & (d < base + NSC)
          idxd[b, j, sl] = jnp.where(inr, d - base, DUMMY)

    def fire(b):
      for j in range(SUP):
        pltpu.async_copy(g_hbm.at[idxs.at[b].at[j]],
                         rows.at[b].at[pl.ds(j * 128, 128)], sems[b])

    def drain(b):
      for j in range(SUP):
        pltpu.make_async_copy(g_hbm.at[idxs.at[b].at[j]],
                              rows.at[b].at[pl.ds(j * 128, 128)],
                              sems[b]).wait()

    def scatter(b, sc):
      left = t_cnt - sc * SUP
      for j in range(SUP):
        @pl.when(j < left)
        def _(j=j):
          pltpu.sync_copy(rows.at[b].at[pl.ds(j * 128, 128)],
                          acc.at[idxd.at[b].at[j]], add=True)

    @pl.when(ns > 0)
    def _():
      load_idx(0, 0)
      fire(0)

    def pair(p, carry):
      for b in range(2):
        sc = 2 * p + b

        @pl.when(sc < ns)
        def _(b=b, sc=sc):
          @pl.when(sc + 1 < ns)
          def _():
            load_idx(1 - b, sc + 1)
            fire(1 - b)

          drain(b)
          scatter(b, sc)
      return carry

    lax.fori_loop(0, (ns + 1) // 2, pair, 0)
    plsc.subcore_barrier()

    for j in range(NB // ZR):
      off = s * NB + j * ZR
      pltpu.sync_copy(acc.at[pl.ds(off, ZR)], stage)
      pltpu.sync_copy(stage, out_hbm.at[pl.ds(base + off, ZR)])

  return pl.kernel(
      body,
      out_type=jax.ShapeDtypeStruct((NP, feat), jnp.float32),
      mesh=mesh,
      compiler_params=pltpu.CompilerParams(use_tc_tiling_on_sc=False),
      scratch_types=[
          pltpu.VMEM((48,), jnp.int32),
          pltpu.VMEM((2, SUP, 128), jnp.int32),
          pltpu.VMEM((2, SUP, 128), jnp.int32),
          pltpu.VMEM((2, SUP * 128, feat), jnp.float32),
          pltpu.VMEM((ZR, feat), jnp.float32),
          pltpu.VMEM_SHARED((NSC + 8, feat), jnp.float32),
          pltpu.SemaphoreType.DMA,
          pltpu.SemaphoreType.DMA,
      ],
  )


def _build_segmax():
  mesh = plsc.VectorSubcoreMesh(core_axis_name="c", subcore_axis_name="s")

  def body(a_hbm, src_hbm, dst_hbm, eoff_hbm, out_hbm, eoff, idxs, idxd,
           rows, acc, sem_a, sem_b):
    c = lax.axis_index("c")
    s = lax.axis_index("s")
    tg = c * 16 + s
    nbase = tg * NB
    sems = (sem_a, sem_b)
    pltpu.sync_copy(eoff_hbm, eoff)
    lo = eoff[pl.ds(tg, 16)][0]
    hi = eoff[pl.ds(tg + 1, 16)][0]
    ninf = jnp.full((16,), NEG_INF, jnp.float32)

    def irow(i, carry):
      for k in range(4):
        acc[i, pl.ds(k * 16, 16)] = ninf
      return carry

    lax.fori_loop(0, NB, irow, 0)
    rlo = lo // 128
    rhi = (hi + 127) // 128
    nr = rhi - rlo

    def load_idx(b, r):
      pltpu.sync_copy(src_hbm.at[pl.ds(r, 1)], idxs.at[b])
      pltpu.sync_copy(dst_hbm.at[pl.ds(r, 1)],
                      idxd.at[b].at[:, pl.ds(0, 128)])

    def fire(b):
      pltpu.async_copy(a_hbm.at[idxs.at[b].at[0]], rows.at[b], sems[b])

    def drain(b):
      pltpu.make_async_copy(a_hbm.at[idxs.at[b].at[0]], rows.at[b],
                            sems[b]).wait()

    def flush(rl_cur, avs):
      for k, ak in enumerate(avs):
        sl = pl.ds(k * 16, 16)
        acc[rl_cur, sl] = jnp.maximum(acc[rl_cur, sl], ak)

    def rmw(b, r):
      jlo = jnp.maximum(lo - r * 128, 0)
      jhi = jnp.minimum(hi - r * 128, 128)

      def edge(j, carry):
        rl_cur = carry[0]
        avs = carry[1:]
        rl = idxd[b, 0, pl.ds(j, 16)][0] - nbase
        new = rl != rl_cur

        @pl.when(new & (rl_cur >= 0))
        def _():
          flush(rl_cur, avs)

        out = [rl]
        for k, ak in enumerate(avs):
          rv = rows[b, j, pl.ds(k * 16, 16)]
          out.append(jnp.maximum(jnp.where(new, ninf, ak), rv))
        return tuple(out)

      carry0 = (jnp.int32(-1), ninf, ninf, ninf, ninf)
      fin = lax.fori_loop(jlo, jhi, edge, carry0)

      @pl.when(fin[0] >= 0)
      def _():
        flush(fin[0], fin[1:])

    @pl.when(nr > 0)
    def _():
      load_idx(0, rlo)
      fire(0)

    def pair(p, carry):
      for b in range(2):
        r = rlo + 2 * p + b

        @pl.when(r < rhi)
        def _(b=b, r=r):
          @pl.when(r + 1 < rhi)
          def _():
            load_idx(1 - b, r + 1)
            fire(1 - b)

          drain(b)
          rmw(b, r)
      return carry

    lax.fori_loop(0, (nr + 1) // 2, pair, 0)
    pltpu.sync_copy(acc, out_hbm.at[pl.ds(nbase, NB)])

  return pl.kernel(
      body,
      out_type=jax.ShapeDtypeStruct((NP, H), jnp.float32),
      mesh=mesh,
      compiler_params=pltpu.CompilerParams(use_tc_tiling_on_sc=False),
      scratch_types=[
          pltpu.VMEM((48,), jnp.int32),
          pltpu.VMEM((2, 1, 128), jnp.int32),
          pltpu.VMEM((2, 1, 144), jnp.int32),
          pltpu.VMEM((2, 128, H), jnp.float32),
          pltpu.VMEM((NB, H), jnp.float32),
          pltpu.SemaphoreType.DMA,
          pltpu.SemaphoreType.DMA,
      ],
  )


_segsum16 = _build_segsum(16)
_segsum32 = _build_segsum(32)
_segmax = _build_segmax()


def _segsum64(g, bsrc2, bdst2, eoff48):
  lo = _segsum32(g[:, :32], bsrc2, bdst2, eoff48)
  hi = _segsum32(g[:, 32:], bsrc2, bdst2, eoff48)
  return jnp.concatenate([lo, hi], axis=1)


def _rowspec(feat):
  return pl.BlockSpec((RB, feat), lambda i: (i, 0))


def _fullspec(shape):
  nd = len(shape)
  return pl.BlockSpec(shape, lambda i, _nd=nd: (0,) * nd)


def _tc_prep(deg16):
  def body(deg_ref, nrm_ref, nsq_ref):
    d = deg_ref[...][:, 0:1]
    nrm = jnp.where(d > 0, lax.rsqrt(jnp.maximum(d, 1.0)), 0.0)
    nrm_ref[...] = nrm
    nsq_ref[...] = nrm * nrm

  return pl.pallas_call(
      body,
      grid=(GRID,),
      in_specs=[_rowspec(16)],
      out_specs=[_rowspec(1), _rowspec(1)],
      out_shape=[jax.ShapeDtypeStruct((NP, 1), jnp.float32)] * 2,
  )(deg16)


def _tc_scale(x, nrm):
  feat = x.shape[1]

  def body(x_ref, nrm_ref, g_ref):
    g_ref[...] = x_ref[...] * nrm_ref[...]

  return pl.pallas_call(
      body,
      grid=(GRID,),
      in_specs=[_rowspec(feat), _rowspec(1)],
      out_specs=_rowspec(feat),
      out_shape=jax.ShapeDtypeStruct((NP, feat), jnp.float32),
  )(x, nrm)


def _tc_mid(s1, nrm, nsq):
  feat = s1.shape[1]

  def body(s_ref, nrm_ref, nsq_ref, xt1_ref, g2_ref):
    sv = s_ref[...]
    xt1_ref[...] = -(sv * nrm_ref[...])
    g2_ref[...] = -(sv * nsq_ref[...])

  return pl.pallas_call(
      body,
      grid=(GRID,),
      in_specs=[_rowspec(feat), _rowspec(1), _rowspec(1)],
      out_specs=[_rowspec(feat), _rowspec(feat)],
      out_shape=[jax.ShapeDtypeStruct((NP, feat), jnp.float32)] * 2,
  )(s1, nrm, nsq)


def _tc_cheb_edge(t0, xt1, s2, nrm, w0, w1, w2, b, twt, pwt, tb, pb):
  feat = t0.shape[1]

  def body(t0_ref, xt1_ref, s2_ref, nrm_ref, w0_ref, w1_ref, w2_ref, b_ref,
           twt_ref, pwt_ref, tb_ref, pb_ref, a_ref, c_ref):
    t0v = t0_ref[...]
    xt2 = -2.0 * (s2_ref[...] * nrm_ref[...]) - t0v
    h = (jnp.dot(t0v, w0_ref[...], preferred_element_type=jnp.float32)
         + jnp.dot(xt1_ref[...], w1_ref[...],
                   preferred_element_type=jnp.float32)
         + jnp.dot(xt2, w2_ref[...], preferred_element_type=jnp.float32)
         + b_ref[...])
    h = jnp.maximum(h, 0.0)
    a_ref[...] = jnp.dot(h, twt_ref[...], preferred_element_type=jnp.float32)
    c_ref[...] = (jnp.dot(h, pwt_ref[...] - twt_ref[...],
                          preferred_element_type=jnp.float32)
                  + tb_ref[...] + pb_ref[...])

  return pl.pallas_call(
      body,
      grid=(GRID,),
      in_specs=[_rowspec(feat), _rowspec(feat), _rowspec(feat), _rowspec(1),
                _fullspec((feat, H)), _fullspec((feat, H)),
                _fullspec((feat, H)), _fullspec((1, H)),
                _fullspec((H, H)), _fullspec((H, H)),
                _fullspec((1, H)), _fullspec((1, H))],
      out_specs=[_rowspec(H), _rowspec(H)],
      out_shape=[jax.ShapeDtypeStruct((NP, H), jnp.float32)] * 2,
  )(t0, xt1, s2, nrm, w0, w1, w2, b, twt, pwt, tb, pb)


def _tc_edge_post(m, cc, nrm):
  def body(m_ref, c_ref, nrm_ref, h_ref, g_ref):
    nc = nrm_ref[...]
    h = jnp.where(nc > 0, jnp.maximum(m_ref[...] + c_ref[...], 0.0), 0.0)
    h_ref[...] = h
    g_ref[...] = h * nc

  return pl.pallas_call(
      body,
      grid=(GRID,),
      in_specs=[_rowspec(H), _rowspec(H), _rowspec(1)],
      out_specs=[_rowspec(H), _rowspec(H)],
      out_shape=[jax.ShapeDtypeStruct((NP, H), jnp.float32)] * 2,
  )(m, cc, nrm)


def _tc_cheb_final(t0, xt1, s2, nrm, w0, w1, w2, b):
  def body(t0_ref, xt1_ref, s2_ref, nrm_ref, w0_ref, w1_ref, w2_ref, b_ref,
           o_ref):
    i = pl.program_id(0)
    t0v = t0_ref[...]
    xt2 = -2.0 * (s2_ref[...] * nrm_ref[...]) - t0v
    h = (jnp.dot(t0v, w0_ref[...], preferred_element_type=jnp.float32)
         + jnp.dot(xt1_ref[...], w1_ref[...],
                   preferred_element_type=jnp.float32)
         + jnp.dot(xt2, w2_ref[...], preferred_element_type=jnp.float32)
         + b_ref[...])
    h = jnp.maximum(h, 0.0)
    ridx = lax.broadcasted_iota(jnp.int32, (RB, 1), 0) + i * RB
    h = jnp.where(ridx < N, h, 0.0)

    @pl.when(i == 0)
    def _():
      o_ref[...] = jnp.zeros_like(o_ref)

    o_ref[...] += jnp.sum(h, axis=0, keepdims=True)

    @pl.when(i == GRID - 1)
    def _():
      o_ref[...] *= (1.0 / N)

  return pl.pallas_call(
      body,
      grid=(GRID,),
      in_specs=[_rowspec(H), _rowspec(H), _rowspec(H), _rowspec(1),
                _fullspec((H, H)), _fullspec((H, H)), _fullspec((H, H)),
                _fullspec((1, H))],
      out_specs=pl.BlockSpec((1, H), lambda i: (0, 0)),
      out_shape=jax.ShapeDtypeStruct((1, H), jnp.float32),
  )(t0, xt1, s2, nrm, w0, w1, w2, b)


def kernel(x, edge_index, W1, b1, W2, b2, W3, b3,
           tw1, tb1, pw1, pb1, tw2, tb2, pw2, pb2):
  src = edge_index[0]
  dst = edge_index[1]

  npad = ERP * 128 - E
  key = (dst.astype(jnp.uint32) << 16) | src.astype(jnp.uint32)
  skey = jnp.sort(key)
  bsrc2 = jnp.concatenate(
      [(skey & 0xFFFF).astype(jnp.int32),
       jnp.zeros((npad,), jnp.int32)]).reshape(ERP, 128)
  bdst2 = jnp.concatenate(
      [(skey >> 16).astype(jnp.int32),
       jnp.full((npad,), NP, jnp.int32)]).reshape(ERP, 128)
  bounds = (jnp.arange(33, dtype=jnp.uint32) * NB) << 16
  eoff = jnp.searchsorted(skey, bounds).astype(jnp.int32)
  eoff48 = jnp.concatenate([eoff, jnp.full((15,), E, jnp.int32)])

  x_p = jnp.pad(x, ((0, NP - N), (0, 0)))
  b1r = b1.reshape(1, H)
  b2r = b2.reshape(1, H)
  b3r = b3.reshape(1, H)
  tb1r = tb1.reshape(1, H)
  pb1r = pb1.reshape(1, H)
  tb2r = tb2.reshape(1, H)
  pb2r = pb2.reshape(1, H)

  ones16 = jnp.ones((NP, 16), jnp.float32)
  deg16 = _segsum16(ones16, bsrc2, bdst2, eoff48)
  nrm, nsq = _tc_prep(deg16)

  g1 = _tc_scale(x_p, nrm)
  s1 = _segsum16(g1, bsrc2, bdst2, eoff48)
  xt1, g2 = _tc_mid(s1, nrm, nsq)
  s2 = _segsum16(g2, bsrc2, bdst2, eoff48)
  a1, c1 = _tc_cheb_edge(x_p, xt1, s2, nrm, W1[0], W1[1], W1[2], b1r,
                         tw1.T, pw1.T, tb1r, pb1r)
  m1 = _segmax(a1, bsrc2, bdst2, eoff48)
  h2, g3 = _tc_edge_post(m1, c1, nrm)

  s3 = _segsum64(g3, bsrc2, bdst2, eoff48)
  xt1b, g4 = _tc_mid(s3, nrm, nsq)
  s4 = _segsum64(g4, bsrc2, bdst2, eoff48)
  a2, c2 = _tc_cheb_edge(h2, xt1b, s4, nrm, W2[0], W2[1], W2[2], b2r,
                         tw2.T, pw2.T, tb2r, pb2r)
  m2 = _segmax(a2, bsrc2, bdst2, eoff48)
  h4, g5 = _tc_edge_post(m2, c2, nrm)

  s5 = _segsum64(g5, bsrc2, bdst2, eoff48)
  xt1c, g6 = _tc_mid(s5, nrm, nsq)
  s6 = _segsum64(g6, bsrc2, bdst2, eoff48)
  return _tc_cheb_final(h4, xt1c, s6, nrm, W3[0], W3[1], W3[2], b3r)
